# Initial kernel scaffold; baseline (speedup 1.0000x reference)
import jax
import jax.numpy as jnp
from jax.experimental import pallas as pl


def kernel(x, edge_index, pos, W_in1, b_in1, W_out1, b_out1, W_in2, b_in2, W_out2, b_out2, W_in3, b_in3, W_out3, b_out3):
    raise NotImplementedError("write your pallas kernel here")


# speedup vs baseline: 5.5188x; 5.5188x over previous
import functools

import jax
import jax.numpy as jnp
from jax import lax
from jax.experimental import pallas as pl
from jax.experimental.pallas import tpu as pltpu
from jax.experimental.pallas import tpu_sc as plsc

N_NODES = 10000
N_EDGES = 160000
IN_CH = 128
HID = 16

NC, NS = 2, 16
NW = NC * NS
EPW = 5120
E_PAD = NW * EPW
EROWS = E_PAD // 128
RPW = EPW // 128
N_PAD = 10240
RPT = N_PAD // NS
EPT = E_PAD // NS


def _sc_mesh():
    return plsc.VectorSubcoreMesh(core_axis_name="c", subcore_axis_name="s")


def _sc_gather1(posx, posy, x, src1, dst1):
    def body(posx_h, posy_h, x_h, src1_h, dst1_h,
             relx_h, rely_h, xg_h,
             posx_v, posy_v, src1_v, dst1_v, relx_v, rely_v,
             rows_v, sem):
        c = lax.axis_index("c")
        s = lax.axis_index("s")
        wid = c * NS + s
        base = wid * EPW
        pltpu.sync_copy(src1_h.at[pl.ds(base, EPW)], src1_v)
        pltpu.sync_copy(dst1_h.at[pl.ds(base, EPW)], dst1_v)
        pltpu.sync_copy(posx_h, posx_v)
        pltpu.sync_copy(posy_h, posy_v)

        def rel_body(i, _):
            s16 = src1_v[pl.ds(i * 16, 16)]
            d16 = dst1_v[pl.ds(i * 16, 16)]
            relx_v[pl.ds(i * 16, 16)] = (plsc.load_gather(posx_v, [s16])
                                         - plsc.load_gather(posx_v, [d16]))
            rely_v[pl.ds(i * 16, 16)] = (plsc.load_gather(posy_v, [s16])
                                         - plsc.load_gather(posy_v, [d16]))
            return 0

        lax.fori_loop(0, EPW // 16, rel_body, 0)
        pltpu.sync_copy(relx_v, relx_h.at[pl.ds(base, EPW)])
        pltpu.sync_copy(rely_v, rely_h.at[pl.ds(base, EPW)])

        def g_body(j, _):
            idx = src1_v.at[pl.ds(j * 128, 128)]
            pltpu.async_copy(x_h.at[idx], rows_v, sem).wait()
            pltpu.sync_copy(rows_v, xg_h.at[pl.ds(base + j * 128, 128)])
            return 0

        lax.fori_loop(0, RPW, g_body, 0)

    f = pl.kernel(
        body,
        out_type=(jax.ShapeDtypeStruct((E_PAD,), jnp.float32),
                  jax.ShapeDtypeStruct((E_PAD,), jnp.float32),
                  jax.ShapeDtypeStruct((E_PAD, IN_CH), jnp.float32)),
        mesh=_sc_mesh(),
        scratch_types=[pltpu.VMEM((N_NODES,), jnp.float32),
                       pltpu.VMEM((N_NODES,), jnp.float32),
                       pltpu.VMEM((EPW,), jnp.int32),
                       pltpu.VMEM((EPW,), jnp.int32),
                       pltpu.VMEM((EPW,), jnp.float32),
                       pltpu.VMEM((EPW,), jnp.float32),
                       pltpu.VMEM((128, IN_CH), jnp.float32),
                       pltpu.SemaphoreType.DMA],
        compiler_params=pltpu.CompilerParams(needs_layout_passes=False),
    )
    return f(posx, posy, x, src1, dst1)


def _sc_gather_h(hT, src1):
    def body(hT_h, src1_h, hgT_h, src1_v, plane_v, col_v, sem):
        c = lax.axis_index("c")
        s = lax.axis_index("s")
        base = s * EPT
        pltpu.sync_copy(src1_h.at[pl.ds(base, EPT)], src1_v)

        for k in range(8):
            pltpu.sync_copy(hT_h.at[c * 8 + k, 0], plane_v)

            def g_body(i, _):
                s16 = src1_v[pl.ds(i * 16, 16)]
                col_v[pl.ds(i * 16, 16)] = plsc.load_gather(plane_v, [s16])
                return 0

            lax.fori_loop(0, EPT // 16, g_body, 0)
            pltpu.sync_copy(col_v, hgT_h.at[c * 8 + k, 0, pl.ds(base, EPT)])

    f = pl.kernel(
        body,
        out_type=jax.ShapeDtypeStruct((16, 1, E_PAD), jnp.float32),
        mesh=_sc_mesh(),
        scratch_types=[pltpu.VMEM((EPT,), jnp.int32),
                       pltpu.VMEM((N_PAD,), jnp.float32),
                       pltpu.VMEM((EPT,), jnp.float32),
                       pltpu.SemaphoreType.DMA],
        compiler_params=pltpu.CompilerParams(needs_layout_passes=False),
    )
    return f(hT, src1)


def _sc_scatter(mT, dstp, binitT):
    def body(mT_h, dstp_h, binitT_h, hT_h,
             dst_v, acc_v, mcol_v, red_v, tmp_v, stage_sh, sem):
        c = lax.axis_index("c")
        s = lax.axis_index("s")
        pltpu.sync_copy(dstp_h.at[pl.ds(s * EPT, EPT)], dst_v)

        for k in range(8):
            pltpu.sync_copy(mT_h.at[c * 8 + k, 0, pl.ds(s * EPT, EPT)],
                            mcol_v)
            zero16 = mcol_v[pl.ds(0, 16)] * 0.0

            def z_body(i, _):
                acc_v[pl.ds(i * 16, 16)] = zero16
                return 0

            lax.fori_loop(0, N_PAD // 16, z_body, 0)

            def s_body(i, _):
                d16 = dst_v[pl.ds(i * 16, 16)]
                v16 = mcol_v[pl.ds(i * 16, 16)]
                plsc.addupdate_scatter(acc_v, [d16], v16)
                return 0

            lax.fori_loop(0, EPT // 16, s_body, 0)
            pltpu.sync_copy(acc_v, stage_sh.at[k, s, 0])
        plsc.subcore_barrier()

        for k in range(8):
            pltpu.sync_copy(binitT_h.at[c * 8 + k, 0, pl.ds(s * RPT, RPT)],
                            red_v)

            def t_body(t, _):
                pltpu.sync_copy(stage_sh.at[k, t, 0, pl.ds(s * RPT, RPT)],
                                tmp_v)

                def r_body(i, _):
                    red_v[pl.ds(i * 16, 16)] = (red_v[pl.ds(i * 16, 16)]
                                                + tmp_v[pl.ds(i * 16, 16)])
                    return 0

                lax.fori_loop(0, RPT // 16, r_body, 0)
                return 0

            lax.fori_loop(0, NS, t_body, 0)
            pltpu.sync_copy(red_v, hT_h.at[c * 8 + k, 0, pl.ds(s * RPT, RPT)])

    f = pl.kernel(
        body,
        out_type=jax.ShapeDtypeStruct((16, 1, N_PAD), jnp.float32),
        mesh=_sc_mesh(),
        scratch_types=[pltpu.VMEM((EPT,), jnp.int32),
                       pltpu.VMEM((N_PAD,), jnp.float32),
                       pltpu.VMEM((EPT,), jnp.float32),
                       pltpu.VMEM((RPT,), jnp.float32),
                       pltpu.VMEM((RPT,), jnp.float32),
                       pltpu.VMEM_SHARED((8, NS, 1, N_PAD), jnp.float32),
                       pltpu.SemaphoreType.DMA],
        compiler_params=pltpu.CompilerParams(needs_layout_passes=False),
    )
    return f(mT, dstp, binitT)


def _tc_dense1(relx, rely, xg, w0, w1, b_, wout):
    T = 512

    def kbody(rx_ref, ry_ref, xg_ref, w0_ref, w1_ref, b_ref, wo_ref, out_ref):
        rx = rx_ref[...]
        ry = ry_ref[...]
        xgt = xg_ref[...]
        xg2 = jnp.concatenate([xgt, xgt], axis=1)
        acc = jnp.zeros((16, T), jnp.float32)
        for kb in range(8):
            sl = pl.ds(kb * 256, 256)
            sp = jnp.maximum(rx * w0_ref[:, sl] + ry * w1_ref[:, sl]
                             + b_ref[:, sl], 0.0)
            y = (sp * xg2).astype(jnp.bfloat16)
            acc = acc + lax.dot_general(wo_ref[sl, :], y,
                                        (((0,), (1,)), ((), ())),
                                        preferred_element_type=jnp.float32)
        out_ref[...] = acc.reshape(16, 1, T)

    return pl.pallas_call(
        kbody,
        grid=(E_PAD // T,),
        in_specs=[pl.BlockSpec((T, 1), lambda i: (i, 0)),
                  pl.BlockSpec((T, 1), lambda i: (i, 0)),
                  pl.BlockSpec((T, IN_CH), lambda i: (i, 0)),
                  pl.BlockSpec((1, 2048), lambda i: (0, 0)),
                  pl.BlockSpec((1, 2048), lambda i: (0, 0)),
                  pl.BlockSpec((1, 2048), lambda i: (0, 0)),
                  pl.BlockSpec((2048, 16), lambda i: (0, 0))],
        out_specs=pl.BlockSpec((16, 1, T), lambda i: (0, 0, i)),
        out_shape=jax.ShapeDtypeStruct((16, 1, E_PAD), jnp.float32),
    )(relx, rely, xg, w0, w1, b_, wout)


def _tc_dense23(relxT, relyT, hgT, w0c, w1c, bc, wout):
    T = 512

    def kbody(rx_ref, ry_ref, hgT_ref, w0_ref, w1_ref, b_ref, wo_ref, out_ref):
        rx = rx_ref[...]
        ry = ry_ref[...]
        hgt = hgT_ref[...].reshape(16, T)
        hg2 = jnp.concatenate([hgt] * 16, axis=0)
        sp = jnp.maximum(w0_ref[...] * rx + w1_ref[...] * ry
                         + b_ref[...], 0.0)
        y = (sp * hg2).astype(jnp.bfloat16)
        acc = lax.dot_general(wo_ref[...], y,
                              (((0,), (0,)), ((), ())),
                              preferred_element_type=jnp.float32)
        out_ref[...] = acc.reshape(16, 1, T)

    return pl.pallas_call(
        kbody,
        grid=(E_PAD // T,),
        in_specs=[pl.BlockSpec((1, T), lambda i: (0, i)),
                  pl.BlockSpec((1, T), lambda i: (0, i)),
                  pl.BlockSpec((16, 1, T), lambda i: (0, 0, i)),
                  pl.BlockSpec((256, 1), lambda i: (0, 0)),
                  pl.BlockSpec((256, 1), lambda i: (0, 0)),
                  pl.BlockSpec((256, 1), lambda i: (0, 0)),
                  pl.BlockSpec((256, 16), lambda i: (0, 0))],
        out_specs=pl.BlockSpec((16, 1, T), lambda i: (0, 0, i)),
        out_shape=jax.ShapeDtypeStruct((16, 1, E_PAD), jnp.float32),
    )(relxT, relyT, hgT, w0c, w1c, bc, wout)


def _permute_in(W, b, C, H):
    Wp = W.reshape(2, C, H).transpose(0, 2, 1).reshape(2, C * H)
    bp = b.reshape(C, H).T.reshape(1, C * H)
    return Wp[0:1], Wp[1:2], bp


def _permute_out(W, C, H):
    return W.reshape(C, H, 16).transpose(1, 0, 2).reshape(C * H, 16).astype(
        jnp.bfloat16)


def kernel(x, edge_index, pos,
           W_in1, b_in1, W_out1, b_out1,
           W_in2, b_in2, W_out2, b_out2,
           W_in3, b_in3, W_out3, b_out3):
    src = edge_index[0].astype(jnp.int32)
    dst = edge_index[1].astype(jnp.int32)
    pad = E_PAD - N_EDGES
    src1 = jnp.concatenate([src, jnp.zeros((pad,), jnp.int32)])
    dst1 = jnp.concatenate([dst, jnp.zeros((pad,), jnp.int32)])
    dstp = jnp.concatenate([dst, jnp.full((pad,), N_NODES, jnp.int32)])
    posx = pos[:, 0]
    posy = pos[:, 1]

    w0_1, w1_1, b1 = _permute_in(W_in1, b_in1, IN_CH, HID)
    wo1 = _permute_out(W_out1, IN_CH, HID)
    w0_2, w1_2, b2 = _permute_in(W_in2, b_in2, HID, HID)
    wo2 = _permute_out(W_out2, HID, HID)
    w0_3, w1_3, b3 = _permute_in(W_in3, b_in3, HID, HID)
    wo3 = _permute_out(W_out3, HID, HID)

    def binit(b_out):
        return jnp.broadcast_to(b_out.reshape(16, 1, 1), (16, 1, N_PAD))


    relx, rely, xg = _sc_gather1(posx, posy, x, src1, dst1)
    relx2 = relx.reshape(E_PAD, 1)
    rely2 = rely.reshape(E_PAD, 1)
    relxT = relx.reshape(1, E_PAD)
    relyT = rely.reshape(1, E_PAD)

    m1 = _tc_dense1(relx2, rely2, xg, w0_1, w1_1, b1, wo1)
    h1 = _sc_scatter(m1, dstp, binit(b_out1))

    hg2 = _sc_gather_h(h1, src1)
    m2 = _tc_dense23(relxT, relyT, hg2,
                     w0_2.reshape(256, 1), w1_2.reshape(256, 1),
                     b2.reshape(256, 1), wo2)
    h2 = _sc_scatter(m2, dstp, binit(b_out2))

    hg3 = _sc_gather_h(h2, src1)
    m3 = _tc_dense23(relxT, relyT, hg3,
                     w0_3.reshape(256, 1), w1_3.reshape(256, 1),
                     b3.reshape(256, 1), wo3)
    h3 = _sc_scatter(m3, dstp, binit(b_out3))

    return jnp.transpose(h3[:, 0, :N_NODES])


# speedup vs baseline: 5.5694x; 1.0092x over previous
import functools

import jax
import jax.numpy as jnp
from jax import lax
from jax.experimental import pallas as pl
from jax.experimental.pallas import tpu as pltpu
from jax.experimental.pallas import tpu_sc as plsc

N_NODES = 10000
N_EDGES = 160000
IN_CH = 128
HID = 16

NC, NS = 2, 16
NW = NC * NS
EPW = 5120
E_PAD = NW * EPW
EROWS = E_PAD // 128
RPW = EPW // 128
N_PAD = 10240
RPT = N_PAD // NS
EPT = E_PAD // NS


def _sc_mesh():
    return plsc.VectorSubcoreMesh(core_axis_name="c", subcore_axis_name="s")


def _sc_gather1(posx, posy, x, src1, dst1):
    def body(posx_h, posy_h, x_h, src1_h, dst1_h,
             relx_h, rely_h, xg_h,
             posx_v, posy_v, src1_v, dst1_v, relx_v, rely_v,
             rows_v, rows2_v, sem, sem2):
        c = lax.axis_index("c")
        s = lax.axis_index("s")
        wid = c * NS + s
        base = wid * EPW
        pltpu.sync_copy(src1_h.at[pl.ds(base, EPW)], src1_v)
        pltpu.sync_copy(dst1_h.at[pl.ds(base, EPW)], dst1_v)
        pltpu.sync_copy(posx_h, posx_v)
        pltpu.sync_copy(posy_h, posy_v)

        def rel_body(i, _):
            s16 = src1_v[pl.ds(i * 16, 16)]
            d16 = dst1_v[pl.ds(i * 16, 16)]
            relx_v[pl.ds(i * 16, 16)] = (plsc.load_gather(posx_v, [s16])
                                         - plsc.load_gather(posx_v, [d16]))
            rely_v[pl.ds(i * 16, 16)] = (plsc.load_gather(posy_v, [s16])
                                         - plsc.load_gather(posy_v, [d16]))
            return 0

        lax.fori_loop(0, EPW // 16, rel_body, 0)
        pltpu.sync_copy(relx_v, relx_h.at[pl.ds(base, EPW)])
        pltpu.sync_copy(rely_v, rely_h.at[pl.ds(base, EPW)])

        def g_body(j2, _):
            idx0 = src1_v.at[pl.ds((2 * j2) * 128, 128)]
            idx1 = src1_v.at[pl.ds((2 * j2 + 1) * 128, 128)]
            cp0 = pltpu.async_copy(x_h.at[idx0], rows_v, sem)
            cp1 = pltpu.async_copy(x_h.at[idx1], rows2_v, sem2)
            cp0.wait()
            pltpu.sync_copy(rows_v,
                            xg_h.at[pl.ds(base + (2 * j2) * 128, 128)])
            cp1.wait()
            pltpu.sync_copy(rows2_v,
                            xg_h.at[pl.ds(base + (2 * j2 + 1) * 128, 128)])
            return 0

        lax.fori_loop(0, RPW // 2, g_body, 0)

    f = pl.kernel(
        body,
        out_type=(jax.ShapeDtypeStruct((E_PAD,), jnp.float32),
                  jax.ShapeDtypeStruct((E_PAD,), jnp.float32),
                  jax.ShapeDtypeStruct((E_PAD, IN_CH), jnp.float32)),
        mesh=_sc_mesh(),
        scratch_types=[pltpu.VMEM((N_NODES,), jnp.float32),
                       pltpu.VMEM((N_NODES,), jnp.float32),
                       pltpu.VMEM((EPW,), jnp.int32),
                       pltpu.VMEM((EPW,), jnp.int32),
                       pltpu.VMEM((EPW,), jnp.float32),
                       pltpu.VMEM((EPW,), jnp.float32),
                       pltpu.VMEM((128, IN_CH), jnp.float32),
                       pltpu.VMEM((128, IN_CH), jnp.float32),
                       pltpu.SemaphoreType.DMA,
                       pltpu.SemaphoreType.DMA],
        compiler_params=pltpu.CompilerParams(needs_layout_passes=False),
    )
    return f(posx, posy, x, src1, dst1)


def _sc_gather_h(hT, src1):
    def body(hT_h, src1_h, hgT_h, src1_v, plane_v, col_v, sem):
        c = lax.axis_index("c")
        s = lax.axis_index("s")
        base = s * EPT
        pltpu.sync_copy(src1_h.at[pl.ds(base, EPT)], src1_v)

        for k in range(8):
            pltpu.sync_copy(hT_h.at[c * 8 + k, 0], plane_v)

            def g_body(i, _):
                s16 = src1_v[pl.ds(i * 16, 16)]
                col_v[pl.ds(i * 16, 16)] = plsc.load_gather(plane_v, [s16])
                return 0

            lax.fori_loop(0, EPT // 16, g_body, 0)
            pltpu.sync_copy(col_v, hgT_h.at[c * 8 + k, 0, pl.ds(base, EPT)])

    f = pl.kernel(
        body,
        out_type=jax.ShapeDtypeStruct((16, 1, E_PAD), jnp.float32),
        mesh=_sc_mesh(),
        scratch_types=[pltpu.VMEM((EPT,), jnp.int32),
                       pltpu.VMEM((N_PAD,), jnp.float32),
                       pltpu.VMEM((EPT,), jnp.float32),
                       pltpu.SemaphoreType.DMA],
        compiler_params=pltpu.CompilerParams(needs_layout_passes=False),
    )
    return f(hT, src1)


def _sc_scatter(mT, dstp, binitT):
    def body(mT_h, dstp_h, binitT_h, hT_h,
             dst_v, acc_v, mcol_v, red_v, tmp_v, stage_sh, sem):
        c = lax.axis_index("c")
        s = lax.axis_index("s")
        pltpu.sync_copy(dstp_h.at[pl.ds(s * EPT, EPT)], dst_v)

        for k in range(8):
            pltpu.sync_copy(mT_h.at[c * 8 + k, 0, pl.ds(s * EPT, EPT)],
                            mcol_v)
            zero16 = mcol_v[pl.ds(0, 16)] * 0.0

            def z_body(i, _):
                acc_v[pl.ds(i * 16, 16)] = zero16
                return 0

            lax.fori_loop(0, N_PAD // 16, z_body, 0)

            def s_body(i, _):
                d16 = dst_v[pl.ds(i * 16, 16)]
                v16 = mcol_v[pl.ds(i * 16, 16)]
                plsc.addupdate_scatter(acc_v, [d16], v16)
                return 0

            lax.fori_loop(0, EPT // 16, s_body, 0)
            pltpu.sync_copy(acc_v, stage_sh.at[k, s, 0])
        plsc.subcore_barrier()

        for k in range(8):
            pltpu.sync_copy(binitT_h.at[c * 8 + k, 0, pl.ds(s * RPT, RPT)],
                            red_v)

            def t_body(t, _):
                pltpu.sync_copy(stage_sh.at[k, t, 0, pl.ds(s * RPT, RPT)],
                                tmp_v)

                def r_body(i, _):
                    red_v[pl.ds(i * 16, 16)] = (red_v[pl.ds(i * 16, 16)]
                                                + tmp_v[pl.ds(i * 16, 16)])
                    return 0

                lax.fori_loop(0, RPT // 16, r_body, 0)
                return 0

            lax.fori_loop(0, NS, t_body, 0)
            pltpu.sync_copy(red_v, hT_h.at[c * 8 + k, 0, pl.ds(s * RPT, RPT)])

    f = pl.kernel(
        body,
        out_type=jax.ShapeDtypeStruct((16, 1, N_PAD), jnp.float32),
        mesh=_sc_mesh(),
        scratch_types=[pltpu.VMEM((EPT,), jnp.int32),
                       pltpu.VMEM((N_PAD,), jnp.float32),
                       pltpu.VMEM((EPT,), jnp.float32),
                       pltpu.VMEM((RPT,), jnp.float32),
                       pltpu.VMEM((RPT,), jnp.float32),
                       pltpu.VMEM_SHARED((8, NS, 1, N_PAD), jnp.float32),
                       pltpu.SemaphoreType.DMA],
        compiler_params=pltpu.CompilerParams(needs_layout_passes=False),
    )
    return f(mT, dstp, binitT)


def _tc_dense1(relx, rely, xg, w0, w1, b_, wout):
    T = 512

    def kbody(rx_ref, ry_ref, xg_ref, w0_ref, w1_ref, b_ref, wo_ref, out_ref):
        rx = rx_ref[...]
        ry = ry_ref[...]
        xgt = xg_ref[...]
        xg2 = jnp.concatenate([xgt, xgt], axis=1)
        acc = jnp.zeros((16, T), jnp.float32)
        for kb in range(8):
            sl = pl.ds(kb * 256, 256)
            sp = jnp.maximum(rx * w0_ref[:, sl] + ry * w1_ref[:, sl]
                             + b_ref[:, sl], 0.0)
            y = (sp * xg2).astype(jnp.bfloat16)
            acc = acc + lax.dot_general(wo_ref[sl, :], y,
                                        (((0,), (1,)), ((), ())),
                                        preferred_element_type=jnp.float32)
        out_ref[...] = acc.reshape(16, 1, T)

    return pl.pallas_call(
        kbody,
        grid=(E_PAD // T,),
        in_specs=[pl.BlockSpec((T, 1), lambda i: (i, 0)),
                  pl.BlockSpec((T, 1), lambda i: (i, 0)),
                  pl.BlockSpec((T, IN_CH), lambda i: (i, 0)),
                  pl.BlockSpec((1, 2048), lambda i: (0, 0)),
                  pl.BlockSpec((1, 2048), lambda i: (0, 0)),
                  pl.BlockSpec((1, 2048), lambda i: (0, 0)),
                  pl.BlockSpec((2048, 16), lambda i: (0, 0))],
        out_specs=pl.BlockSpec((16, 1, T), lambda i: (0, 0, i)),
        out_shape=jax.ShapeDtypeStruct((16, 1, E_PAD), jnp.float32),
    )(relx, rely, xg, w0, w1, b_, wout)


def _tc_dense23(relxT, relyT, hgT, w0c, w1c, bc, wout):
    T = 512

    def kbody(rx_ref, ry_ref, hgT_ref, w0_ref, w1_ref, b_ref, wo_ref, out_ref):
        rx = rx_ref[...]
        ry = ry_ref[...]
        hgt = hgT_ref[...].reshape(16, T)
        hg2 = jnp.concatenate([hgt] * 16, axis=0)
        sp = jnp.maximum(w0_ref[...] * rx + w1_ref[...] * ry
                         + b_ref[...], 0.0)
        y = (sp * hg2).astype(jnp.bfloat16)
        acc = lax.dot_general(wo_ref[...], y,
                              (((0,), (0,)), ((), ())),
                              preferred_element_type=jnp.float32)
        out_ref[...] = acc.reshape(16, 1, T)

    return pl.pallas_call(
        kbody,
        grid=(E_PAD // T,),
        in_specs=[pl.BlockSpec((1, T), lambda i: (0, i)),
                  pl.BlockSpec((1, T), lambda i: (0, i)),
                  pl.BlockSpec((16, 1, T), lambda i: (0, 0, i)),
                  pl.BlockSpec((256, 1), lambda i: (0, 0)),
                  pl.BlockSpec((256, 1), lambda i: (0, 0)),
                  pl.BlockSpec((256, 1), lambda i: (0, 0)),
                  pl.BlockSpec((256, 16), lambda i: (0, 0))],
        out_specs=pl.BlockSpec((16, 1, T), lambda i: (0, 0, i)),
        out_shape=jax.ShapeDtypeStruct((16, 1, E_PAD), jnp.float32),
    )(relxT, relyT, hgT, w0c, w1c, bc, wout)


def _permute_in(W, b, C, H):
    Wp = W.reshape(2, C, H).transpose(0, 2, 1).reshape(2, C * H)
    bp = b.reshape(C, H).T.reshape(1, C * H)
    return Wp[0:1], Wp[1:2], bp


def _permute_out(W, C, H):
    return W.reshape(C, H, 16).transpose(1, 0, 2).reshape(C * H, 16).astype(
        jnp.bfloat16)


def kernel(x, edge_index, pos,
           W_in1, b_in1, W_out1, b_out1,
           W_in2, b_in2, W_out2, b_out2,
           W_in3, b_in3, W_out3, b_out3):
    src = edge_index[0].astype(jnp.int32)
    dst = edge_index[1].astype(jnp.int32)
    pad = E_PAD - N_EDGES
    src1 = jnp.concatenate([src, jnp.zeros((pad,), jnp.int32)])
    dst1 = jnp.concatenate([dst, jnp.zeros((pad,), jnp.int32)])
    dstp = jnp.concatenate([dst, jnp.full((pad,), N_NODES, jnp.int32)])
    posx = pos[:, 0]
    posy = pos[:, 1]

    w0_1, w1_1, b1 = _permute_in(W_in1, b_in1, IN_CH, HID)
    wo1 = _permute_out(W_out1, IN_CH, HID)
    w0_2, w1_2, b2 = _permute_in(W_in2, b_in2, HID, HID)
    wo2 = _permute_out(W_out2, HID, HID)
    w0_3, w1_3, b3 = _permute_in(W_in3, b_in3, HID, HID)
    wo3 = _permute_out(W_out3, HID, HID)

    def binit(b_out):
        return jnp.broadcast_to(b_out.reshape(16, 1, 1), (16, 1, N_PAD))


    relx, rely, xg = _sc_gather1(posx, posy, x, src1, dst1)
    relx2 = relx.reshape(E_PAD, 1)
    rely2 = rely.reshape(E_PAD, 1)
    relxT = relx.reshape(1, E_PAD)
    relyT = rely.reshape(1, E_PAD)

    m1 = _tc_dense1(relx2, rely2, xg, w0_1, w1_1, b1, wo1)
    h1 = _sc_scatter(m1, dstp, binit(b_out1))

    hg2 = _sc_gather_h(h1, src1)
    m2 = _tc_dense23(relxT, relyT, hg2,
                     w0_2.reshape(256, 1), w1_2.reshape(256, 1),
                     b2.reshape(256, 1), wo2)
    h2 = _sc_scatter(m2, dstp, binit(b_out2))

    hg3 = _sc_gather_h(h2, src1)
    m3 = _tc_dense23(relxT, relyT, hg3,
                     w0_3.reshape(256, 1), w1_3.reshape(256, 1),
                     b3.reshape(256, 1), wo3)
    h3 = _sc_scatter(m3, dstp, binit(b_out3))

    return jnp.transpose(h3[:, 0, :N_NODES])
